# baseline (speedup 1.0000x reference)
import jax
import jax.numpy as jnp
from jax import lax
from jax.experimental import pallas as pl
from jax.experimental.pallas import tpu as pltpu
from jax.experimental.pallas import tpu_sc as plsc

N = 10000
E = 320000
D_EDGE = 16
D = 128
DH = D // 2
NUM_GRAPHS = 64
NUM_LAYERS = 3

NC = 2
NS = 16
C = 64
E_PAD = 327680
EPT = E_PAD // NS
NCH = EPT // C
NHF = 2
NCH_H = NCH // NHF
N_PAD = 10112
ROWS_PER_TILE = N_PAD // NS


_BE = 2560


def _ea_body(attr_ref, w_ref, b_ref, out_ref):
    out_ref[...] = lax.dot_general(
        attr_ref[...], w_ref[...], (((1,), (1,)), ((), ())),
        preferred_element_type=jnp.float32) + b_ref[...]


def _compute_ea(edge_attr, w, b_row):
    return pl.pallas_call(
        _ea_body,
        grid=(E // _BE,),
        in_specs=[
            pl.BlockSpec((_BE, D_EDGE), lambda e: (e, 0)),
            pl.BlockSpec((D, D_EDGE), lambda e: (0, 0)),
            pl.BlockSpec((1, D), lambda e: (0, 0)),
        ],
        out_specs=pl.BlockSpec((_BE, D), lambda e: (e, 0)),
        out_shape=jax.ShapeDtypeStruct((E_PAD, D), jnp.float32),
    )(edge_attr, w, b_row)


def _mp_body(h_hbm, ea_hbm, src_hbm, dst_hbm, zeros_hbm, agg_hbm,
             srcv, dstv, eav, gatv, msgv, hsh, aggsh, ea_sem, g_sem, s_sem):
    c = lax.axis_index("c")
    s = lax.axis_index("s")

    r0 = s * ROWS_PER_TILE
    pltpu.sync_copy(h_hbm.at[c, pl.ds(r0, ROWS_PER_TILE)],
                    hsh.at[pl.ds(r0, ROWS_PER_TILE)])
    pltpu.sync_copy(zeros_hbm.at[pl.ds(r0, ROWS_PER_TILE)],
                    aggsh.at[pl.ds(r0, ROWS_PER_TILE)])
    plsc.subcore_barrier()

    ebase = s * EPT

    def in_start(j, b, hf):
        e0 = ebase + (hf * NCH_H + j) * C
        pltpu.async_copy(ea_hbm.at[pl.ds(e0, C), pl.ds(c * DH, DH)],
                         eav.at[b], ea_sem.at[b])
        pltpu.async_copy(hsh.at[srcv.at[j]], gatv.at[b], g_sem.at[b])

    def in_wait(j, b, hf):
        e0 = ebase + (hf * NCH_H + j) * C
        pltpu.make_async_copy(ea_hbm.at[pl.ds(e0, C), pl.ds(c * DH, DH)],
                              eav.at[b], ea_sem.at[b]).wait()
        pltpu.make_async_copy(hsh.at[srcv.at[j]], gatv.at[b],
                              g_sem.at[b]).wait()

    def s_start(j, b):
        pltpu.async_copy(msgv.at[b], aggsh.at[dstv.at[j]], s_sem.at[b],
                         add=True)

    def s_wait(j, b):
        pltpu.make_async_copy(msgv.at[b], aggsh.at[dstv.at[j]],
                              s_sem.at[b]).wait()

    for hf in range(NHF):
        pltpu.sync_copy(src_hbm.at[s, hf], srcv)
        pltpu.sync_copy(dst_hbm.at[s, hf], dstv)

        in_start(0, 0, hf)
        in_start(1, 1, hf)

        @pl.loop(0, NCH_H, step=2)
        def _(i):
            for b in range(2):
                j = i + b
                in_wait(j, b, hf)

                @pl.when(j >= 2)
                def _():
                    s_wait(j - 2, b)

                @plsc.parallel_loop(0, C, unroll=8)
                def _(r):
                    for k in range(DH // 16):
                        sl = pl.ds(k * 16, 16)
                        msgv[b, r, sl] = jnp.maximum(
                            gatv[b, r, sl] + eav[b, r, sl], 0.0)

                @pl.when(j + 2 < NCH_H)
                def _():
                    in_start(j + 2, b, hf)

                s_start(j, b)

        s_wait(NCH_H - 2, 0)
        s_wait(NCH_H - 1, 1)

    plsc.subcore_barrier()

    pltpu.sync_copy(aggsh.at[pl.ds(r0, ROWS_PER_TILE)],
                    agg_hbm.at[c, pl.ds(r0, ROWS_PER_TILE)])


def _message_pass(h2, ea_l, src_rs, dst_rs, zeros_nd):
    mesh = plsc.VectorSubcoreMesh(core_axis_name="c", subcore_axis_name="s",
                                  num_cores=NC, num_subcores=NS)
    kern = pl.kernel(
        _mp_body,
        out_type=jax.ShapeDtypeStruct((NC, N_PAD, DH), jnp.float32),
        mesh=mesh,
        scratch_types=[
            pltpu.VMEM((NCH_H, C), jnp.int32),
            pltpu.VMEM((NCH_H, C), jnp.int32),
            pltpu.VMEM((2, C, DH), jnp.float32),
            pltpu.VMEM((2, C, DH), jnp.float32),
            pltpu.VMEM((2, C, DH), jnp.float32),
            pltpu.VMEM_SHARED((N_PAD, DH), jnp.float32),
            pltpu.VMEM_SHARED((N_PAD, DH), jnp.float32),
            pltpu.SemaphoreType.DMA((2,)),
            pltpu.SemaphoreType.DMA((2,)),
            pltpu.SemaphoreType.DMA((2,)),
        ],
        compiler_params=pltpu.CompilerParams(use_tc_tiling_on_sc=False),
    )
    return kern(h2, ea_l, src_rs, dst_rs, zeros_nd)


_BN = 2000


def _mlp_body(h_ref, agg_ref, wt_ref, b_ref, out_ref):
    hp_lo = h_ref[0] + agg_ref[0]
    hp_hi = h_ref[1] + agg_ref[1]
    hp = jnp.concatenate([hp_lo, hp_hi], axis=1)
    acc = jnp.dot(hp, wt_ref[...],
                  preferred_element_type=jnp.float32) + b_ref[...]
    acc = jnp.maximum(acc, 0.0)
    out_ref[0] = acc[:, :DH]
    out_ref[1] = acc[:, DH:]


def _node_mlp(h2, agg, wt, b_row):
    return pl.pallas_call(
        _mlp_body,
        grid=(N // _BN,),
        in_specs=[
            pl.BlockSpec((NC, _BN, DH), lambda i: (0, i, 0)),
            pl.BlockSpec((NC, _BN, DH), lambda i: (0, i, 0)),
            pl.BlockSpec((D, D), lambda i: (0, 0)),
            pl.BlockSpec((1, D), lambda i: (0, 0)),
        ],
        out_specs=pl.BlockSpec((NC, _BN, DH), lambda i: (0, i, 0)),
        out_shape=jax.ShapeDtypeStruct((NC, N_PAD, DH), jnp.float32),
    )(h2, agg, wt, b_row)


def _split_body(x_ref, out_ref):
    out_ref[0] = x_ref[:, :DH]
    out_ref[1] = x_ref[:, DH:]


def _split_x(x):
    return pl.pallas_call(
        _split_body,
        grid=(N // _BN,),
        in_specs=[pl.BlockSpec((_BN, D), lambda i: (i, 0))],
        out_specs=pl.BlockSpec((NC, _BN, DH), lambda i: (0, i, 0)),
        out_shape=jax.ShapeDtypeStruct((NC, N_PAD, DH), jnp.float32),
    )(x)


def _pool_body(batch_ref, h_ref, out_ref, sums, counts):
    i = pl.program_id(0)

    @pl.when(i == 0)
    def _():
        sums[...] = jnp.zeros_like(sums)
        counts[...] = jnp.zeros_like(counts)

    b = batch_ref[...]
    gids = lax.broadcasted_iota(jnp.int32, (1, NUM_GRAPHS), 1)
    onehot = (b == gids).astype(jnp.float32)
    h = jnp.concatenate([h_ref[0], h_ref[1]], axis=1)
    sums[...] += lax.dot_general(onehot, h, (((0,), (0,)), ((), ())),
                                 preferred_element_type=jnp.float32)
    ones = jnp.ones((_BN, 1), jnp.float32)
    counts[...] += lax.dot_general(onehot, ones, (((0,), (0,)), ((), ())),
                                   preferred_element_type=jnp.float32)

    @pl.when(i == N // _BN - 1)
    def _():
        out_ref[...] = sums[...] / jnp.maximum(counts[...], 1.0)


def _mean_pool(batch_col, h2):
    return pl.pallas_call(
        _pool_body,
        grid=(N // _BN,),
        in_specs=[
            pl.BlockSpec((_BN, 1), lambda i: (i, 0)),
            pl.BlockSpec((NC, _BN, DH), lambda i: (0, i, 0)),
        ],
        out_specs=pl.BlockSpec((NUM_GRAPHS, D), lambda i: (0, 0)),
        out_shape=jax.ShapeDtypeStruct((NUM_GRAPHS, D), jnp.float32),
        scratch_shapes=[
            pltpu.VMEM((NUM_GRAPHS, D), jnp.float32),
            pltpu.VMEM((NUM_GRAPHS, 1), jnp.float32),
        ],
    )(batch_col, h2)


def kernel(x, edge_index, edge_attr, batch,
           lin_e_W_0, lin_e_b_0, nn_W_0, nn_b_0,
           lin_e_W_1, lin_e_b_1, nn_W_1, nn_b_1,
           lin_e_W_2, lin_e_b_2, nn_W_2, nn_b_2):
    lin_ws = [lin_e_W_0, lin_e_W_1, lin_e_W_2]
    lin_bs = [lin_e_b_0[None, :], lin_e_b_1[None, :], lin_e_b_2[None, :]]
    nn_wts = [nn_W_0.T, nn_W_1.T, nn_W_2.T]
    nn_bs = [nn_b_0[None, :], nn_b_1[None, :], nn_b_2[None, :]]

    npad = E_PAD - E
    pad_ids = jnp.arange(npad, dtype=jnp.int32)
    src_pad = jnp.concatenate([edge_index[0], pad_ids % N])
    dst_pad = jnp.concatenate([edge_index[1], N + pad_ids % (N_PAD - N)])
    src_rs = src_pad.reshape(NS, NHF, NCH_H, C)
    dst_rs = dst_pad.reshape(NS, NHF, NCH_H, C)
    zeros_nd = jnp.zeros((N_PAD, DH), jnp.float32)

    h2 = _split_x(x)
    for l in range(NUM_LAYERS):
        ea_l = _compute_ea(edge_attr, lin_ws[l], lin_bs[l])
        agg = _message_pass(h2, ea_l, src_rs, dst_rs, zeros_nd)
        h2 = _node_mlp(h2, agg, nn_wts[l], nn_bs[l])

    return _mean_pool(batch.reshape(N, 1), h2)


# speedup vs baseline: 1.0037x; 1.0037x over previous
import jax
import jax.numpy as jnp
from jax import lax
from jax.experimental import pallas as pl
from jax.experimental.pallas import tpu as pltpu
from jax.experimental.pallas import tpu_sc as plsc

N = 10000
E = 320000
D_EDGE = 16
D = 128
DH = D // 2
NUM_GRAPHS = 64
NUM_LAYERS = 3

NC = 2
NS = 16
C = 64
E_PAD = 327680
EPT = E_PAD // NS
NCH = EPT // C
NHF = 8
NCH_H = NCH // NHF
N_PAD = 10112
ROWS_PER_TILE = N_PAD // NS


_BE = 2560


def _ea_body(attr_ref, w_ref, b_ref, out_ref):
    out_ref[...] = lax.dot_general(
        attr_ref[...], w_ref[...], (((1,), (1,)), ((), ())),
        preferred_element_type=jnp.float32) + b_ref[...]


def _compute_ea(edge_attr, w, b_row):
    return pl.pallas_call(
        _ea_body,
        grid=(E // _BE,),
        in_specs=[
            pl.BlockSpec((_BE, D_EDGE), lambda e: (e, 0)),
            pl.BlockSpec((D, D_EDGE), lambda e: (0, 0)),
            pl.BlockSpec((1, D), lambda e: (0, 0)),
        ],
        out_specs=pl.BlockSpec((_BE, D), lambda e: (e, 0)),
        out_shape=jax.ShapeDtypeStruct((E_PAD, D), jnp.float32),
    )(edge_attr, w, b_row)


def _mp_body(h_hbm, ea_hbm, src_hbm, dst_hbm, zeros_hbm, agg_hbm,
             srcv, dstv, eav, gatv, msgv, hsh, aggsh, ea_sem, g_sem, s_sem):
    c = lax.axis_index("c")
    s = lax.axis_index("s")

    r0 = s * ROWS_PER_TILE
    pltpu.sync_copy(h_hbm.at[c, pl.ds(r0, ROWS_PER_TILE)],
                    hsh.at[pl.ds(r0, ROWS_PER_TILE)])
    pltpu.sync_copy(zeros_hbm.at[pl.ds(r0, ROWS_PER_TILE)],
                    aggsh.at[pl.ds(r0, ROWS_PER_TILE)])
    plsc.subcore_barrier()

    ebase = s * EPT

    def in_start(j, b, hf):
        e0 = ebase + (hf * NCH_H + j) * C
        pltpu.async_copy(ea_hbm.at[pl.ds(e0, C), pl.ds(c * DH, DH)],
                         eav.at[b], ea_sem.at[b])
        pltpu.async_copy(hsh.at[srcv.at[j]], gatv.at[b], g_sem.at[b])

    def in_wait(j, b, hf):
        e0 = ebase + (hf * NCH_H + j) * C
        pltpu.make_async_copy(ea_hbm.at[pl.ds(e0, C), pl.ds(c * DH, DH)],
                              eav.at[b], ea_sem.at[b]).wait()
        pltpu.make_async_copy(hsh.at[srcv.at[j]], gatv.at[b],
                              g_sem.at[b]).wait()

    def s_start(j, b):
        pltpu.async_copy(msgv.at[b], aggsh.at[dstv.at[j]], s_sem.at[b],
                         add=True)

    def s_wait(j, b):
        pltpu.make_async_copy(msgv.at[b], aggsh.at[dstv.at[j]],
                              s_sem.at[b]).wait()

    for hf in range(NHF):
        pltpu.sync_copy(src_hbm.at[s, hf], srcv)
        pltpu.sync_copy(dst_hbm.at[s, hf], dstv)

        for b in range(4):
            in_start(b, b, hf)

        @pl.loop(0, NCH_H, step=4)
        def _(i):
            for b in range(4):
                j = i + b
                m = b % 2
                in_wait(j, b, hf)

                @pl.when(j >= 2)
                def _():
                    s_wait(j - 2, m)

                @plsc.parallel_loop(0, C, unroll=8)
                def _(r):
                    for k in range(DH // 16):
                        sl = pl.ds(k * 16, 16)
                        msgv[m, r, sl] = jnp.maximum(
                            gatv[b, r, sl] + eav[b, r, sl], 0.0)

                @pl.when(j + 4 < NCH_H)
                def _():
                    in_start(j + 4, b, hf)

                s_start(j, m)

        s_wait(NCH_H - 2, 0)
        s_wait(NCH_H - 1, 1)

    plsc.subcore_barrier()

    pltpu.sync_copy(aggsh.at[pl.ds(r0, ROWS_PER_TILE)],
                    agg_hbm.at[c, pl.ds(r0, ROWS_PER_TILE)])


def _message_pass(h2, ea_l, src_rs, dst_rs, zeros_nd):
    mesh = plsc.VectorSubcoreMesh(core_axis_name="c", subcore_axis_name="s",
                                  num_cores=NC, num_subcores=NS)
    kern = pl.kernel(
        _mp_body,
        out_type=jax.ShapeDtypeStruct((NC, N_PAD, DH), jnp.float32),
        mesh=mesh,
        scratch_types=[
            pltpu.VMEM((NCH_H, C), jnp.int32),
            pltpu.VMEM((NCH_H, C), jnp.int32),
            pltpu.VMEM((4, C, DH), jnp.float32),
            pltpu.VMEM((4, C, DH), jnp.float32),
            pltpu.VMEM((2, C, DH), jnp.float32),
            pltpu.VMEM_SHARED((N_PAD, DH), jnp.float32),
            pltpu.VMEM_SHARED((N_PAD, DH), jnp.float32),
            pltpu.SemaphoreType.DMA((4,)),
            pltpu.SemaphoreType.DMA((4,)),
            pltpu.SemaphoreType.DMA((2,)),
        ],
        compiler_params=pltpu.CompilerParams(use_tc_tiling_on_sc=False),
    )
    return kern(h2, ea_l, src_rs, dst_rs, zeros_nd)


_BN = 2000


def _mlp_body(h_ref, agg_ref, wt_ref, b_ref, out_ref):
    hp_lo = h_ref[0] + agg_ref[0]
    hp_hi = h_ref[1] + agg_ref[1]
    hp = jnp.concatenate([hp_lo, hp_hi], axis=1)
    acc = jnp.dot(hp, wt_ref[...],
                  preferred_element_type=jnp.float32) + b_ref[...]
    acc = jnp.maximum(acc, 0.0)
    out_ref[0] = acc[:, :DH]
    out_ref[1] = acc[:, DH:]


def _node_mlp(h2, agg, wt, b_row):
    return pl.pallas_call(
        _mlp_body,
        grid=(N // _BN,),
        in_specs=[
            pl.BlockSpec((NC, _BN, DH), lambda i: (0, i, 0)),
            pl.BlockSpec((NC, _BN, DH), lambda i: (0, i, 0)),
            pl.BlockSpec((D, D), lambda i: (0, 0)),
            pl.BlockSpec((1, D), lambda i: (0, 0)),
        ],
        out_specs=pl.BlockSpec((NC, _BN, DH), lambda i: (0, i, 0)),
        out_shape=jax.ShapeDtypeStruct((NC, N_PAD, DH), jnp.float32),
    )(h2, agg, wt, b_row)


def _split_body(x_ref, out_ref):
    out_ref[0] = x_ref[:, :DH]
    out_ref[1] = x_ref[:, DH:]


def _split_x(x):
    return pl.pallas_call(
        _split_body,
        grid=(N // _BN,),
        in_specs=[pl.BlockSpec((_BN, D), lambda i: (i, 0))],
        out_specs=pl.BlockSpec((NC, _BN, DH), lambda i: (0, i, 0)),
        out_shape=jax.ShapeDtypeStruct((NC, N_PAD, DH), jnp.float32),
    )(x)


def _pool_body(batch_ref, h_ref, out_ref, sums, counts):
    i = pl.program_id(0)

    @pl.when(i == 0)
    def _():
        sums[...] = jnp.zeros_like(sums)
        counts[...] = jnp.zeros_like(counts)

    b = batch_ref[...]
    gids = lax.broadcasted_iota(jnp.int32, (1, NUM_GRAPHS), 1)
    onehot = (b == gids).astype(jnp.float32)
    h = jnp.concatenate([h_ref[0], h_ref[1]], axis=1)
    sums[...] += lax.dot_general(onehot, h, (((0,), (0,)), ((), ())),
                                 preferred_element_type=jnp.float32)
    ones = jnp.ones((_BN, 1), jnp.float32)
    counts[...] += lax.dot_general(onehot, ones, (((0,), (0,)), ((), ())),
                                   preferred_element_type=jnp.float32)

    @pl.when(i == N // _BN - 1)
    def _():
        out_ref[...] = sums[...] / jnp.maximum(counts[...], 1.0)


def _mean_pool(batch_col, h2):
    return pl.pallas_call(
        _pool_body,
        grid=(N // _BN,),
        in_specs=[
            pl.BlockSpec((_BN, 1), lambda i: (i, 0)),
            pl.BlockSpec((NC, _BN, DH), lambda i: (0, i, 0)),
        ],
        out_specs=pl.BlockSpec((NUM_GRAPHS, D), lambda i: (0, 0)),
        out_shape=jax.ShapeDtypeStruct((NUM_GRAPHS, D), jnp.float32),
        scratch_shapes=[
            pltpu.VMEM((NUM_GRAPHS, D), jnp.float32),
            pltpu.VMEM((NUM_GRAPHS, 1), jnp.float32),
        ],
    )(batch_col, h2)


def kernel(x, edge_index, edge_attr, batch,
           lin_e_W_0, lin_e_b_0, nn_W_0, nn_b_0,
           lin_e_W_1, lin_e_b_1, nn_W_1, nn_b_1,
           lin_e_W_2, lin_e_b_2, nn_W_2, nn_b_2):
    lin_ws = [lin_e_W_0, lin_e_W_1, lin_e_W_2]
    lin_bs = [lin_e_b_0[None, :], lin_e_b_1[None, :], lin_e_b_2[None, :]]
    nn_wts = [nn_W_0.T, nn_W_1.T, nn_W_2.T]
    nn_bs = [nn_b_0[None, :], nn_b_1[None, :], nn_b_2[None, :]]

    npad = E_PAD - E
    pad_ids = jnp.arange(npad, dtype=jnp.int32)
    src_pad = jnp.concatenate([edge_index[0], pad_ids % N])
    dst_pad = jnp.concatenate([edge_index[1], N + pad_ids % (N_PAD - N)])
    src_rs = src_pad.reshape(NS, NHF, NCH_H, C)
    dst_rs = dst_pad.reshape(NS, NHF, NCH_H, C)
    zeros_nd = jnp.zeros((N_PAD, DH), jnp.float32)

    h2 = _split_x(x)
    for l in range(NUM_LAYERS):
        ea_l = _compute_ea(edge_attr, lin_ws[l], lin_bs[l])
        agg = _message_pass(h2, ea_l, src_rs, dst_rs, zeros_nd)
        h2 = _node_mlp(h2, agg, nn_wts[l], nn_bs[l])

    return _mean_pool(batch.reshape(N, 1), h2)
